# baseline (speedup 1.0000x reference)
import functools

import jax
import jax.numpy as jnp
from jax import lax
from jax.experimental import pallas as pl
from jax.experimental.pallas import tpu as pltpu
from jax.experimental.pallas import tpu_sc as plsc

NC = 2
NS = 16
NW = NC * NS
L = 16
EB = 128
DH = 64


def _rsqrt16(x):
    i = lax.bitcast_convert_type(x, jnp.int32)
    i = jnp.int32(0x5F3759DF) - (i >> 1)
    y = lax.bitcast_convert_type(i, jnp.float32)
    for _ in range(3):
        y = y * (1.5 - 0.5 * x * y * y)
    return y


def _wid(c, s):
    return s * NC + c


def _deg_body(nb, n_pad, dst_hbm, degpart_hbm, dst_v, ones_v, zrow_v, acc):
    c = lax.axis_index("c")
    s = lax.axis_index("s")
    w = _wid(c, s)
    rps = n_pad // NS
    for i in range(EB // L):
        ones_v[pl.ds(i * L, L)] = jnp.ones((L,), jnp.float32)
    for i in range(128 // L):
        zrow_v[pl.ds(i * L, L)] = jnp.zeros((L,), jnp.float32)
    for t in range(rps // 128):
        pltpu.sync_copy(zrow_v, acc.at[pl.ds(s * rps + t * 128, 128)])
    pltpu.sync_copy(dst_hbm.at[pl.ds(w * nb, nb)], dst_v)
    plsc.subcore_barrier()

    def body(j, carry):
        pltpu.sync_copy(ones_v, acc.at[dst_v.at[j]], add=True)
        return carry

    lax.fori_loop(0, nb, body, 0)
    plsc.subcore_barrier()
    pltpu.sync_copy(acc.at[pl.ds(s * rps, rps)],
                    degpart_hbm.at[pl.ds(c * n_pad + s * rps, rps)])


def _scale0_body(n_pad, degpart_hbm, xa_hbm, xb_hbm, g0a_hbm, g0b_hbm,
                 norm_hbm, inv_hbm, d0, d1, nrm, inv, xa, xb):
    c = lax.axis_index("c")
    s = lax.axis_index("s")
    rpw = n_pad // NW
    rb = _wid(c, s) * rpw
    pltpu.sync_copy(degpart_hbm.at[pl.ds(rb, rpw)], d0)
    pltpu.sync_copy(degpart_hbm.at[pl.ds(n_pad + rb, rpw)], d1)
    pltpu.sync_copy(xa_hbm.at[pl.ds(rb, rpw)], xa)
    pltpu.sync_copy(xb_hbm.at[pl.ds(rb, rpw)], xb)
    for t in range(rpw // L):
        deg = jnp.maximum(d0[pl.ds(t * L, L)] + d1[pl.ds(t * L, L)], 1.0)
        nrm[pl.ds(t * L, L)] = _rsqrt16(deg)
        inv[pl.ds(t * L, L)] = 1.0 / deg
    pltpu.sync_copy(nrm, norm_hbm.at[pl.ds(rb, rpw)])
    pltpu.sync_copy(inv, inv_hbm.at[pl.ds(rb, rpw)])

    def row(r, carry):
        b = plsc.load_gather(nrm, [jnp.full((L,), r, jnp.int32)])
        for cb in range(DH // L):
            sl = pl.ds(cb * L, L)
            xa[r, sl] = xa[r, sl] * b
            xb[r, sl] = xb[r, sl] * b
        return carry

    lax.fori_loop(0, rpw, row, 0)
    pltpu.sync_copy(xa, g0a_hbm.at[pl.ds(rb, rpw)])
    pltpu.sync_copy(xb, g0b_hbm.at[pl.ds(rb, rpw)])


def _hop_body(nb, n_pad, g_hbm, src_hbm, dst_hbm, zeros_hbm, parts_hbm,
              src_v, dst_v, rows_a, rows_b, acc, gsem):
    c = lax.axis_index("c")
    s = lax.axis_index("s")
    w = _wid(c, s)
    rps = n_pad // NS
    pltpu.sync_copy(zeros_hbm.at[pl.ds(s * rps, rps)],
                    acc.at[pl.ds(s * rps, rps)])
    pltpu.sync_copy(src_hbm.at[pl.ds(w * nb, nb)], src_v)
    pltpu.sync_copy(dst_hbm.at[pl.ds(w * nb, nb)], dst_v)
    plsc.subcore_barrier()

    bufs = (rows_a, rows_b)

    def fire(j, p):
        pltpu.async_copy(g_hbm.at[src_v.at[j]], bufs[p], gsem)

    def drain(j, p):
        pltpu.make_async_copy(g_hbm.at[src_v.at[j]], bufs[p], gsem).wait()

    def scatter(j, p):
        pltpu.sync_copy(bufs[p], acc.at[dst_v.at[j]], add=True)

    CH = 8

    def outer(gg, carry):
        base = gg * CH
        fire(base, 0)
        for k in range(CH - 1):
            drain(base + k, k % 2)
            fire(base + k + 1, (k + 1) % 2)
            scatter(base + k, k % 2)
        drain(base + CH - 1, (CH - 1) % 2)
        scatter(base + CH - 1, (CH - 1) % 2)
        return carry

    lax.fori_loop(0, nb // CH, outer, 0)
    plsc.subcore_barrier()
    pltpu.sync_copy(acc.at[pl.ds(s * rps, rps)],
                    parts_hbm.at[pl.ds(c * n_pad + s * rps, rps)])


def _combine_body(n_pad, pa_hbm, pb_hbm, inv_hbm, ga_hbm, gb_hbm, p0, p1, iv):
    c = lax.axis_index("c")
    s = lax.axis_index("s")
    rpw = n_pad // NW
    rb = _wid(c, s) * rpw
    pltpu.sync_copy(inv_hbm.at[pl.ds(rb, rpw)], iv)
    for parts_hbm, g_hbm in ((pa_hbm, ga_hbm), (pb_hbm, gb_hbm)):
        pltpu.sync_copy(parts_hbm.at[pl.ds(rb, rpw)], p0)
        pltpu.sync_copy(parts_hbm.at[pl.ds(n_pad + rb, rpw)], p1)

        def row(r, carry):
            b = plsc.load_gather(iv, [jnp.full((L,), r, jnp.int32)])
            for cb in range(DH // L):
                sl = pl.ds(cb * L, L)
                p0[r, sl] = (p0[r, sl] + p1[r, sl]) * b
            return carry

        lax.fori_loop(0, rpw, row, 0)
        pltpu.sync_copy(p0, g_hbm.at[pl.ds(rb, rpw)])


def _mm_body(pa0_ref, pa1_ref, pb0_ref, pb1_ref, n_ref, w_ref, o_ref):
    ha = pa0_ref[...] + pa1_ref[...]
    hb = pb0_ref[...] + pb1_ref[...]
    h = jnp.concatenate([ha, hb], axis=1) * n_ref[...]
    o_ref[...] = lax.dot_general(h, w_ref[...], (((1,), (1,)), ((), ())),
                                 preferred_element_type=jnp.float32)


@jax.jit
def kernel(x, edge_index, W):
    n, d = x.shape
    e = edge_index.shape[1]
    n_pad = ((n + 511) // 512) * 512
    nb = (e + NW * EB - 1) // (NW * EB)
    nb = ((nb + 7) // 8) * 8
    e_pad = NW * nb * EB
    npad_extra = n_pad - n

    src = edge_index[0]
    dst = edge_index[1]
    pe = e_pad - e
    pad_src = jnp.arange(pe, dtype=jnp.int32) % jnp.int32(n)
    pad_dst = jnp.int32(n) + jnp.arange(pe, dtype=jnp.int32) % jnp.int32(npad_extra)
    srcp = jnp.concatenate([src, pad_src]).reshape(NW * nb, EB)
    dstp = jnp.concatenate([dst, pad_dst]).reshape(NW * nb, EB)
    xp = jnp.pad(x, ((0, npad_extra), (0, 0)))
    xpa = xp[:, :DH]
    xpb = xp[:, DH:]
    zeros2d = jnp.zeros((n_pad, DH), jnp.float32)

    mesh = plsc.VectorSubcoreMesh(core_axis_name="c", subcore_axis_name="s")
    params = pltpu.CompilerParams(needs_layout_passes=False,
                                  use_tc_tiling_on_sc=False)

    deg_call = pl.kernel(
        functools.partial(_deg_body, nb, n_pad),
        out_type=jax.ShapeDtypeStruct((NC * n_pad,), jnp.float32),
        mesh=mesh,
        compiler_params=params,
        scratch_types=[
            pltpu.VMEM((nb, EB), jnp.int32),
            pltpu.VMEM((EB,), jnp.float32),
            pltpu.VMEM((128,), jnp.float32),
            pltpu.VMEM_SHARED((n_pad,), jnp.float32),
        ],
    )
    degpart = deg_call(dstp)

    scale0_call = pl.kernel(
        functools.partial(_scale0_body, n_pad),
        out_type=(
            jax.ShapeDtypeStruct((n_pad, DH), jnp.float32),
            jax.ShapeDtypeStruct((n_pad, DH), jnp.float32),
            jax.ShapeDtypeStruct((n_pad,), jnp.float32),
            jax.ShapeDtypeStruct((n_pad,), jnp.float32),
        ),
        mesh=mesh,
        compiler_params=params,
        scratch_types=[
            pltpu.VMEM((n_pad // NW,), jnp.float32),
            pltpu.VMEM((n_pad // NW,), jnp.float32),
            pltpu.VMEM((n_pad // NW,), jnp.float32),
            pltpu.VMEM((n_pad // NW,), jnp.float32),
            pltpu.VMEM((n_pad // NW, DH), jnp.float32),
            pltpu.VMEM((n_pad // NW, DH), jnp.float32),
        ],
    )
    g0a, g0b, nrm, inv = scale0_call(degpart, xpa, xpb)

    hop_call = pl.kernel(
        functools.partial(_hop_body, nb, n_pad),
        out_type=jax.ShapeDtypeStruct((NC * n_pad, DH), jnp.float32),
        mesh=mesh,
        compiler_params=params,
        scratch_types=[
            pltpu.VMEM((nb, EB), jnp.int32),
            pltpu.VMEM((nb, EB), jnp.int32),
            pltpu.VMEM((EB, DH), jnp.float32),
            pltpu.VMEM((EB, DH), jnp.float32),
            pltpu.VMEM_SHARED((n_pad, DH), jnp.float32),
            pltpu.SemaphoreType.DMA,
        ],
    )
    parts1a = hop_call(g0a, srcp, dstp, zeros2d)
    parts1b = hop_call(g0b, srcp, dstp, zeros2d)

    combine_call = pl.kernel(
        functools.partial(_combine_body, n_pad),
        out_type=(
            jax.ShapeDtypeStruct((n_pad, DH), jnp.float32),
            jax.ShapeDtypeStruct((n_pad, DH), jnp.float32),
        ),
        mesh=mesh,
        compiler_params=params,
        scratch_types=[
            pltpu.VMEM((n_pad // NW, DH), jnp.float32),
            pltpu.VMEM((n_pad // NW, DH), jnp.float32),
            pltpu.VMEM((n_pad // NW,), jnp.float32),
        ],
    )
    g1a, g1b = combine_call(parts1a, parts1b, inv)

    parts2a = hop_call(g1a, srcp, dstp, zeros2d)
    parts2b = hop_call(g1b, srcp, dstp, zeros2d)

    blk = 1024
    mm_call = pl.pallas_call(
        _mm_body,
        grid=(n_pad // blk,),
        in_specs=[
            pl.BlockSpec((blk, DH), lambda i: (i, 0)),
            pl.BlockSpec((blk, DH), lambda i: (i, 0)),
            pl.BlockSpec((blk, DH), lambda i: (i, 0)),
            pl.BlockSpec((blk, DH), lambda i: (i, 0)),
            pl.BlockSpec((blk, 1), lambda i: (i, 0)),
            pl.BlockSpec((d, d), lambda i: (0, 0)),
        ],
        out_specs=pl.BlockSpec((blk, d), lambda i: (i, 0)),
        out_shape=jax.ShapeDtypeStruct((n_pad, d), jnp.float32),
    )
    out = mm_call(parts2a[:n_pad], parts2a[n_pad:], parts2b[:n_pad],
                  parts2b[n_pad:], nrm.reshape(n_pad, 1), W)
    return out[:n]


# speedup vs baseline: 1.1007x; 1.1007x over previous
import functools

import jax
import jax.numpy as jnp
from jax import lax
from jax.experimental import pallas as pl
from jax.experimental.pallas import tpu as pltpu
from jax.experimental.pallas import tpu_sc as plsc

NC = 2
NS = 16
NW = NC * NS
L = 16
EB = 128


def _rsqrt16(x):
    i = lax.bitcast_convert_type(x, jnp.int32)
    i = jnp.int32(0x5F3759DF) - (i >> 1)
    y = lax.bitcast_convert_type(i, jnp.float32)
    for _ in range(3):
        y = y * (1.5 - 0.5 * x * y * y)
    return y


def _wid(c, s):
    return s * NC + c


def _deg_body(nb, n_pad, dst_hbm, degpart_hbm, dst_v, ones_v, zrow_v, acc):
    c = lax.axis_index("c")
    s = lax.axis_index("s")
    w = _wid(c, s)
    rps = n_pad // NS
    for i in range(EB // L):
        ones_v[pl.ds(i * L, L)] = jnp.ones((L,), jnp.float32)
    for i in range(128 // L):
        zrow_v[pl.ds(i * L, L)] = jnp.zeros((L,), jnp.float32)
    for t in range(rps // 128):
        pltpu.sync_copy(zrow_v, acc.at[pl.ds(s * rps + t * 128, 128)])
    pltpu.sync_copy(dst_hbm.at[pl.ds(w * nb, nb)], dst_v)
    plsc.subcore_barrier()

    def body(j, carry):
        pltpu.sync_copy(ones_v, acc.at[dst_v.at[j]], add=True)
        return carry

    lax.fori_loop(0, nb, body, 0)
    plsc.subcore_barrier()
    pltpu.sync_copy(acc.at[pl.ds(s * rps, rps)],
                    degpart_hbm.at[pl.ds(c * n_pad + s * rps, rps)])


def _hop_body(nb, n_pad, g_hbm, src_hbm, dst_hbm, zeros_hbm, parts_hbm,
              src_v, dst_v, rows_a, acc, gsem):
    c = lax.axis_index("c")
    s = lax.axis_index("s")
    w = _wid(c, s)
    rps = n_pad // NS
    pltpu.sync_copy(zeros_hbm.at[pl.ds(s * rps, rps)],
                    acc.at[pl.ds(s * rps, rps)])
    pltpu.sync_copy(src_hbm.at[pl.ds(w * nb, nb)], src_v)
    pltpu.sync_copy(dst_hbm.at[pl.ds(w * nb, nb)], dst_v)
    plsc.subcore_barrier()

    def body(j, carry):
        pltpu.async_copy(g_hbm.at[src_v.at[j]], rows_a, gsem).wait()
        pltpu.sync_copy(rows_a, acc.at[dst_v.at[j]], add=True)
        return carry

    lax.fori_loop(0, nb, body, 0)
    plsc.subcore_barrier()
    pltpu.sync_copy(acc.at[pl.ds(s * rps, rps)],
                    parts_hbm.at[pl.ds(c * n_pad + s * rps, rps)])


def _scale0_tc_body(d0_ref, d1_ref, x_ref, g0_ref, nrm_ref, inv_ref):
    deg = jnp.maximum(d0_ref[...] + d1_ref[...], 1.0)
    nrm = lax.rsqrt(deg)
    nrm_ref[...] = nrm
    inv_ref[...] = 1.0 / deg
    g0_ref[...] = x_ref[...] * nrm


def _combine_tc_body(p0_ref, p1_ref, inv_ref, g_ref):
    g_ref[...] = (p0_ref[...] + p1_ref[...]) * inv_ref[...]


def _mm_body(p0_ref, p1_ref, n_ref, w_ref, o_ref):
    h = (p0_ref[...] + p1_ref[...]) * n_ref[...]
    o_ref[...] = lax.dot_general(h, w_ref[...], (((1,), (1,)), ((), ())),
                                 preferred_element_type=jnp.float32)


@jax.jit
def kernel(x, edge_index, W):
    n, d = x.shape
    e = edge_index.shape[1]
    n_pad = ((n + 511) // 512) * 512
    nb = (e + NW * EB - 1) // (NW * EB)
    nb = ((nb + 7) // 8) * 8
    e_pad = NW * nb * EB
    npad_extra = n_pad - n

    src = edge_index[0]
    dst = edge_index[1]
    pe = e_pad - e
    pad_src = jnp.arange(pe, dtype=jnp.int32) % jnp.int32(n)
    pad_dst = jnp.int32(n) + jnp.arange(pe, dtype=jnp.int32) % jnp.int32(npad_extra)
    srcp = jnp.concatenate([src, pad_src]).reshape(NW * nb, EB)
    dstp = jnp.concatenate([dst, pad_dst]).reshape(NW * nb, EB)
    xp = jnp.pad(x, ((0, npad_extra), (0, 0)))
    zeros2d = jnp.zeros((n_pad, d), jnp.float32)

    mesh = plsc.VectorSubcoreMesh(core_axis_name="c", subcore_axis_name="s")

    deg_call = pl.kernel(
        functools.partial(_deg_body, nb, n_pad),
        out_type=jax.ShapeDtypeStruct((NC * n_pad,), jnp.float32),
        mesh=mesh,
        compiler_params=pltpu.CompilerParams(needs_layout_passes=False),
        scratch_types=[
            pltpu.VMEM((nb, EB), jnp.int32),
            pltpu.VMEM((EB,), jnp.float32),
            pltpu.VMEM((128,), jnp.float32),
            pltpu.VMEM_SHARED((n_pad,), jnp.float32),
        ],
    )
    degpart = deg_call(dstp)

    blk = 1024
    scale0_call = pl.pallas_call(
        _scale0_tc_body,
        grid=(n_pad // blk,),
        in_specs=[
            pl.BlockSpec((blk, 1), lambda i: (i, 0)),
            pl.BlockSpec((blk, 1), lambda i: (i, 0)),
            pl.BlockSpec((blk, d), lambda i: (i, 0)),
        ],
        out_specs=(
            pl.BlockSpec((blk, d), lambda i: (i, 0)),
            pl.BlockSpec((blk, 1), lambda i: (i, 0)),
            pl.BlockSpec((blk, 1), lambda i: (i, 0)),
        ),
        out_shape=(
            jax.ShapeDtypeStruct((n_pad, d), jnp.float32),
            jax.ShapeDtypeStruct((n_pad, 1), jnp.float32),
            jax.ShapeDtypeStruct((n_pad, 1), jnp.float32),
        ),
    )
    g0, nrm2d, inv2d = scale0_call(degpart[:n_pad].reshape(n_pad, 1),
                                   degpart[n_pad:].reshape(n_pad, 1), xp)

    hop_call = pl.kernel(
        functools.partial(_hop_body, nb, n_pad),
        out_type=jax.ShapeDtypeStruct((NC * n_pad, d), jnp.float32),
        mesh=mesh,
        compiler_params=pltpu.CompilerParams(needs_layout_passes=False),
        scratch_types=[
            pltpu.VMEM((nb, EB), jnp.int32),
            pltpu.VMEM((nb, EB), jnp.int32),
            pltpu.VMEM((EB, d), jnp.float32),
            pltpu.VMEM_SHARED((n_pad, d), jnp.float32),
            pltpu.SemaphoreType.DMA,
        ],
    )
    parts1 = hop_call(g0, srcp, dstp, zeros2d)

    combine_call = pl.pallas_call(
        _combine_tc_body,
        grid=(n_pad // blk,),
        in_specs=[
            pl.BlockSpec((blk, d), lambda i: (i, 0)),
            pl.BlockSpec((blk, d), lambda i: (i, 0)),
            pl.BlockSpec((blk, 1), lambda i: (i, 0)),
        ],
        out_specs=pl.BlockSpec((blk, d), lambda i: (i, 0)),
        out_shape=jax.ShapeDtypeStruct((n_pad, d), jnp.float32),
    )
    g1 = combine_call(parts1[:n_pad], parts1[n_pad:], inv2d)

    parts2 = hop_call(g1, srcp, dstp, zeros2d)

    mm_call = pl.pallas_call(
        _mm_body,
        grid=(n_pad // blk,),
        in_specs=[
            pl.BlockSpec((blk, d), lambda i: (i, 0)),
            pl.BlockSpec((blk, d), lambda i: (i, 0)),
            pl.BlockSpec((blk, 1), lambda i: (i, 0)),
            pl.BlockSpec((d, d), lambda i: (0, 0)),
        ],
        out_specs=pl.BlockSpec((blk, d), lambda i: (i, 0)),
        out_shape=jax.ShapeDtypeStruct((n_pad, d), jnp.float32),
    )
    out = mm_call(parts2[:n_pad], parts2[n_pad:], nrm2d, W)
    return out[:n]


# speedup vs baseline: 1.1323x; 1.0287x over previous
import functools

import jax
import jax.numpy as jnp
from jax import lax
from jax.experimental import pallas as pl
from jax.experimental.pallas import tpu as pltpu
from jax.experimental.pallas import tpu_sc as plsc

NC = 2
NS = 16
NW = NC * NS
L = 16
EB = 128


def _rsqrt16(x):
    i = lax.bitcast_convert_type(x, jnp.int32)
    i = jnp.int32(0x5F3759DF) - (i >> 1)
    y = lax.bitcast_convert_type(i, jnp.float32)
    for _ in range(3):
        y = y * (1.5 - 0.5 * x * y * y)
    return y


def _wid(c, s):
    return s * NC + c


def _deg_body(nb, n_pad, dst_hbm, degpart_hbm, dst_v, ones_v, zrow_v, acc):
    c = lax.axis_index("c")
    s = lax.axis_index("s")
    w = _wid(c, s)
    rps = n_pad // NS
    for i in range(EB // L):
        ones_v[pl.ds(i * L, L)] = jnp.ones((L,), jnp.float32)
    for i in range(128 // L):
        zrow_v[pl.ds(i * L, L)] = jnp.zeros((L,), jnp.float32)
    for t in range(rps // 128):
        pltpu.sync_copy(zrow_v, acc.at[pl.ds(s * rps + t * 128, 128)])
    pltpu.sync_copy(dst_hbm.at[pl.ds(w * nb, nb)], dst_v)
    plsc.subcore_barrier()

    def body(j, carry):
        pltpu.sync_copy(ones_v, acc.at[dst_v.at[j]], add=True)
        return carry

    lax.fori_loop(0, nb, body, 0)
    plsc.subcore_barrier()
    pltpu.sync_copy(acc.at[pl.ds(s * rps, rps)],
                    degpart_hbm.at[pl.ds(c * n_pad + s * rps, rps)])


def _hop_body(nb, n_pad, g_hbm, src_hbm, dst_hbm, zeros_hbm, parts_hbm,
              src_v, dst_v, rows_a, acc, gsem):
    c = lax.axis_index("c")
    s = lax.axis_index("s")
    w = _wid(c, s)
    rps = n_pad // NS
    pltpu.sync_copy(zeros_hbm.at[pl.ds(s * rps, rps)],
                    acc.at[pl.ds(s * rps, rps)])
    pltpu.sync_copy(src_hbm.at[pl.ds(w * nb, nb)], src_v)
    pltpu.sync_copy(dst_hbm.at[pl.ds(w * nb, nb)], dst_v)
    plsc.subcore_barrier()

    def body(j, carry):
        pltpu.async_copy(g_hbm.at[src_v.at[j]], rows_a, gsem).wait()
        pltpu.sync_copy(rows_a, acc.at[dst_v.at[j]], add=True)
        return carry

    lax.fori_loop(0, nb, body, 0)
    plsc.subcore_barrier()
    pltpu.sync_copy(acc.at[pl.ds(s * rps, rps)],
                    parts_hbm.at[pl.ds(c * n_pad + s * rps, rps)])


def _scale0_tc_body(d0_ref, d1_ref, x_ref, g0_ref, nrm_ref, inv_ref):
    deg = jnp.maximum(d0_ref[...] + d1_ref[...], 1.0)
    nrm = lax.rsqrt(deg)
    nrm_ref[...] = nrm
    inv_ref[...] = 1.0 / deg
    g0_ref[...] = x_ref[...] * nrm


def _combine_tc_body(p0_ref, p1_ref, inv_ref, g_ref):
    g_ref[...] = (p0_ref[...] + p1_ref[...]) * inv_ref[...]


def _mm_body(p0_ref, p1_ref, n_ref, w_ref, o_ref):
    h = (p0_ref[...] + p1_ref[...]) * n_ref[...]
    o_ref[...] = lax.dot_general(h, w_ref[...], (((1,), (1,)), ((), ())),
                                 preferred_element_type=jnp.float32)


@jax.jit
def kernel(x, edge_index, W):
    n, d = x.shape
    e = edge_index.shape[1]
    n_pad = ((n + 511) // 512) * 512
    nb = (e + NW * EB - 1) // (NW * EB)
    nb = ((nb + 7) // 8) * 8
    e_pad = NW * nb * EB
    npad_extra = n_pad - n

    src = edge_index[0]
    dst = edge_index[1]
    pe = e_pad - e
    pad_src = jnp.arange(pe, dtype=jnp.int32) % jnp.int32(n)
    pad_dst = jnp.int32(n) + jnp.arange(pe, dtype=jnp.int32) % jnp.int32(npad_extra)
    srcp = jnp.concatenate([src, pad_src]).reshape(NW * nb, EB)
    dstp = jnp.concatenate([dst, pad_dst]).reshape(NW * nb, EB)
    xp = jnp.pad(x, ((0, npad_extra), (0, 0)))
    zeros2d = jnp.zeros((n_pad, d), jnp.float32)

    mesh = plsc.VectorSubcoreMesh(core_axis_name="c", subcore_axis_name="s")

    deg_call = pl.kernel(
        functools.partial(_deg_body, nb, n_pad),
        out_type=jax.ShapeDtypeStruct((NC * n_pad,), jnp.float32),
        mesh=mesh,
        compiler_params=pltpu.CompilerParams(needs_layout_passes=False),
        scratch_types=[
            pltpu.VMEM((nb, EB), jnp.int32),
            pltpu.VMEM((EB,), jnp.float32),
            pltpu.VMEM((128,), jnp.float32),
            pltpu.VMEM_SHARED((n_pad,), jnp.float32),
        ],
    )
    degpart = deg_call(dstp)

    blk = 1024
    scale0_call = pl.pallas_call(
        _scale0_tc_body,
        grid=(n_pad // blk,),
        in_specs=[
            pl.BlockSpec((blk, 1), lambda i: (i, 0)),
            pl.BlockSpec((blk, 1), lambda i: (i, 0)),
            pl.BlockSpec((blk, d), lambda i: (i, 0)),
        ],
        out_specs=(
            pl.BlockSpec((blk, d), lambda i: (i, 0)),
            pl.BlockSpec((blk, 1), lambda i: (i, 0)),
            pl.BlockSpec((blk, 1), lambda i: (i, 0)),
        ),
        out_shape=(
            jax.ShapeDtypeStruct((n_pad, d), jnp.float32),
            jax.ShapeDtypeStruct((n_pad, 1), jnp.float32),
            jax.ShapeDtypeStruct((n_pad, 1), jnp.float32),
        ),
    )
    g0, nrm2d, inv2d = scale0_call(degpart[:n_pad].reshape(n_pad, 1),
                                   degpart[n_pad:].reshape(n_pad, 1), xp)

    hop_call = pl.kernel(
        functools.partial(_hop_body, nb, n_pad),
        out_type=jax.ShapeDtypeStruct((NC * n_pad, d), jnp.float32),
        mesh=mesh,
        compiler_params=pltpu.CompilerParams(needs_layout_passes=False),
        scratch_types=[
            pltpu.VMEM((nb, EB), jnp.int32),
            pltpu.VMEM((nb, EB), jnp.int32),
            pltpu.VMEM((EB, d), jnp.float32),
            pltpu.VMEM_SHARED((n_pad, d), jnp.float32),
            pltpu.SemaphoreType.DMA,
        ],
    )
    parts1 = hop_call(g0, srcp, dstp, zeros2d)

    nblk = n_pad // blk
    combine_call = pl.pallas_call(
        _combine_tc_body,
        grid=(nblk,),
        in_specs=[
            pl.BlockSpec((blk, d), lambda i: (i, 0)),
            pl.BlockSpec((blk, d), lambda i: (i + nblk, 0)),
            pl.BlockSpec((blk, 1), lambda i: (i, 0)),
        ],
        out_specs=pl.BlockSpec((blk, d), lambda i: (i, 0)),
        out_shape=jax.ShapeDtypeStruct((n_pad, d), jnp.float32),
    )
    g1 = combine_call(parts1, parts1, inv2d)

    parts2 = hop_call(g1, srcp, dstp, zeros2d)

    mm_call = pl.pallas_call(
        _mm_body,
        grid=(nblk,),
        in_specs=[
            pl.BlockSpec((blk, d), lambda i: (i, 0)),
            pl.BlockSpec((blk, d), lambda i: (i + nblk, 0)),
            pl.BlockSpec((blk, 1), lambda i: (i, 0)),
            pl.BlockSpec((d, d), lambda i: (0, 0)),
        ],
        out_specs=pl.BlockSpec((blk, d), lambda i: (i, 0)),
        out_shape=jax.ShapeDtypeStruct((n_pad, d), jnp.float32),
    )
    out = mm_call(parts2, parts2, nrm2d, W)
    return out[:n]


# speedup vs baseline: 1.1477x; 1.0136x over previous
import functools

import jax
import jax.numpy as jnp
from jax import lax
from jax.experimental import pallas as pl
from jax.experimental.pallas import tpu as pltpu
from jax.experimental.pallas import tpu_sc as plsc

NC = 2
NS = 16
NW = NC * NS
L = 16
EB = 128


def _rsqrt16(x):
    i = lax.bitcast_convert_type(x, jnp.int32)
    i = jnp.int32(0x5F3759DF) - (i >> 1)
    y = lax.bitcast_convert_type(i, jnp.float32)
    for _ in range(3):
        y = y * (1.5 - 0.5 * x * y * y)
    return y


def _wid(c, s):
    return s * NC + c


def _deg_body(nb, n_pad, dst_hbm, degpart_hbm, dst_v, ones_v, zrow_v, acc):
    c = lax.axis_index("c")
    s = lax.axis_index("s")
    w = _wid(c, s)
    rps = n_pad // NS
    for i in range(EB // L):
        ones_v[pl.ds(i * L, L)] = jnp.ones((L,), jnp.float32)
    for i in range(128 // L):
        zrow_v[pl.ds(i * L, L)] = jnp.zeros((L,), jnp.float32)
    for t in range(rps // 128):
        pltpu.sync_copy(zrow_v, acc.at[pl.ds(s * rps + t * 128, 128)])
    pltpu.sync_copy(dst_hbm.at[pl.ds(w * nb, nb)], dst_v)
    plsc.subcore_barrier()

    def body(j, carry):
        pltpu.sync_copy(ones_v, acc.at[dst_v.at[j]], add=True)
        return carry

    lax.fori_loop(0, nb, body, 0)
    plsc.subcore_barrier()
    pltpu.sync_copy(acc.at[pl.ds(s * rps, rps)],
                    degpart_hbm.at[pl.ds(c * n_pad + s * rps, rps)])


def _hop_body(nb, n_pad, g_hbm, src_hbm, dst_hbm, zeros_hbm, parts_hbm,
              src_v, dst_v, rows_a, acc, gsem):
    c = lax.axis_index("c")
    s = lax.axis_index("s")
    w = _wid(c, s)
    rps = n_pad // NS
    pltpu.sync_copy(zeros_hbm.at[pl.ds(s * rps, rps)],
                    acc.at[pl.ds(s * rps, rps)])
    pltpu.sync_copy(src_hbm.at[pl.ds(w * nb, nb)], src_v)
    pltpu.sync_copy(dst_hbm.at[pl.ds(w * nb, nb)], dst_v)
    plsc.subcore_barrier()

    def body(j, carry):
        pltpu.async_copy(g_hbm.at[src_v.at[j]], rows_a, gsem).wait()
        pltpu.sync_copy(rows_a, acc.at[dst_v.at[j]], add=True)
        return carry

    lax.fori_loop(0, nb, body, 0)
    plsc.subcore_barrier()
    pltpu.sync_copy(acc.at[pl.ds(s * rps, rps)],
                    parts_hbm.at[pl.ds(c * n_pad + s * rps, rps)])


def _scale0_tc_body(d0_ref, d1_ref, x_ref, g0_ref, nrm_ref, inv_ref):
    deg = jnp.maximum(d0_ref[...] + d1_ref[...], 1.0)
    nrm = lax.rsqrt(deg)
    nrm_ref[...] = nrm
    inv_ref[...] = 1.0 / deg
    g0_ref[...] = x_ref[...] * nrm


def _combine_tc_body(p0_ref, p1_ref, inv_ref, g_ref):
    g_ref[...] = (p0_ref[...] + p1_ref[...]) * inv_ref[...]


def _mm_body(p0_ref, p1_ref, n_ref, w_ref, o_ref):
    h = (p0_ref[...] + p1_ref[...]) * n_ref[...]
    o_ref[...] = lax.dot_general(h, w_ref[...], (((1,), (1,)), ((), ())),
                                 preferred_element_type=jnp.float32)


@jax.jit
def kernel(x, edge_index, W):
    n, d = x.shape
    e = edge_index.shape[1]
    n_pad = ((n + 511) // 512) * 512
    nb = (e + NW * EB - 1) // (NW * EB)
    nb = ((nb + 7) // 8) * 8
    e_pad = NW * nb * EB
    npad_extra = n_pad - n

    src = edge_index[0]
    dst = edge_index[1]
    pe = e_pad - e
    pad_src = jnp.arange(pe, dtype=jnp.int32) % jnp.int32(n)
    pad_dst = jnp.int32(n) + jnp.arange(pe, dtype=jnp.int32) % jnp.int32(npad_extra)
    srcp = jnp.concatenate([src, pad_src]).reshape(NW * nb, EB)
    dstp = jnp.concatenate([dst, pad_dst]).reshape(NW * nb, EB)
    xp = jnp.pad(x, ((0, npad_extra), (0, 0)))
    zeros2d = jnp.zeros((n_pad, d), jnp.float32)

    mesh = plsc.VectorSubcoreMesh(core_axis_name="c", subcore_axis_name="s")

    deg_call = pl.kernel(
        functools.partial(_deg_body, nb, n_pad),
        out_type=jax.ShapeDtypeStruct((NC * n_pad,), jnp.float32),
        mesh=mesh,
        compiler_params=pltpu.CompilerParams(needs_layout_passes=False),
        scratch_types=[
            pltpu.VMEM((nb, EB), jnp.int32),
            pltpu.VMEM((EB,), jnp.float32),
            pltpu.VMEM((128,), jnp.float32),
            pltpu.VMEM_SHARED((n_pad,), jnp.float32),
        ],
    )
    degpart = deg_call(dstp)

    blk = 2048
    scale0_call = pl.pallas_call(
        _scale0_tc_body,
        grid=(n_pad // blk,),
        in_specs=[
            pl.BlockSpec((blk, 1), lambda i: (i, 0)),
            pl.BlockSpec((blk, 1), lambda i: (i, 0)),
            pl.BlockSpec((blk, d), lambda i: (i, 0)),
        ],
        out_specs=(
            pl.BlockSpec((blk, d), lambda i: (i, 0)),
            pl.BlockSpec((blk, 1), lambda i: (i, 0)),
            pl.BlockSpec((blk, 1), lambda i: (i, 0)),
        ),
        out_shape=(
            jax.ShapeDtypeStruct((n_pad, d), jnp.float32),
            jax.ShapeDtypeStruct((n_pad, 1), jnp.float32),
            jax.ShapeDtypeStruct((n_pad, 1), jnp.float32),
        ),
    )
    g0, nrm2d, inv2d = scale0_call(degpart[:n_pad].reshape(n_pad, 1),
                                   degpart[n_pad:].reshape(n_pad, 1), xp)

    hop_call = pl.kernel(
        functools.partial(_hop_body, nb, n_pad),
        out_type=jax.ShapeDtypeStruct((NC * n_pad, d), jnp.float32),
        mesh=mesh,
        compiler_params=pltpu.CompilerParams(needs_layout_passes=False),
        scratch_types=[
            pltpu.VMEM((nb, EB), jnp.int32),
            pltpu.VMEM((nb, EB), jnp.int32),
            pltpu.VMEM((EB, d), jnp.float32),
            pltpu.VMEM_SHARED((n_pad, d), jnp.float32),
            pltpu.SemaphoreType.DMA,
        ],
    )
    parts1 = hop_call(g0, srcp, dstp, zeros2d)

    nblk = n_pad // blk
    combine_call = pl.pallas_call(
        _combine_tc_body,
        grid=(nblk,),
        in_specs=[
            pl.BlockSpec((blk, d), lambda i: (i, 0)),
            pl.BlockSpec((blk, d), lambda i: (i + nblk, 0)),
            pl.BlockSpec((blk, 1), lambda i: (i, 0)),
        ],
        out_specs=pl.BlockSpec((blk, d), lambda i: (i, 0)),
        out_shape=jax.ShapeDtypeStruct((n_pad, d), jnp.float32),
    )
    g1 = combine_call(parts1, parts1, inv2d)

    parts2 = hop_call(g1, srcp, dstp, zeros2d)

    mm_call = pl.pallas_call(
        _mm_body,
        grid=(nblk,),
        in_specs=[
            pl.BlockSpec((blk, d), lambda i: (i, 0)),
            pl.BlockSpec((blk, d), lambda i: (i + nblk, 0)),
            pl.BlockSpec((blk, 1), lambda i: (i, 0)),
            pl.BlockSpec((d, d), lambda i: (0, 0)),
        ],
        out_specs=pl.BlockSpec((blk, d), lambda i: (i, 0)),
        out_shape=jax.ShapeDtypeStruct((n_pad, d), jnp.float32),
    )
    out = mm_call(parts2, parts2, nrm2d, W)
    return out[:n]


# speedup vs baseline: 1.1486x; 1.0008x over previous
import functools

import jax
import jax.numpy as jnp
from jax import lax
from jax.experimental import pallas as pl
from jax.experimental.pallas import tpu as pltpu
from jax.experimental.pallas import tpu_sc as plsc

NC = 2
NS = 16
NW = NC * NS
L = 16
EB = 128


def _wid(c, s):
    return s * NC + c


def _deg_body(nb, n_pad, dst_hbm, degpart_hbm, dst_v, ones_v, zrow_v, acc):
    c = lax.axis_index("c")
    s = lax.axis_index("s")
    w = _wid(c, s)
    rps = n_pad // NS
    for i in range(EB // L):
        ones_v[pl.ds(i * L, L)] = jnp.ones((L,), jnp.float32)
    for i in range(128 // L):
        zrow_v[pl.ds(i * L, L)] = jnp.zeros((L,), jnp.float32)
    for t in range(rps // 128):
        pltpu.sync_copy(zrow_v, acc.at[pl.ds(s * rps + t * 128, 128)])
    pltpu.sync_copy(dst_hbm.at[pl.ds(w * nb, nb)], dst_v)
    plsc.subcore_barrier()

    def body(j, carry):
        pltpu.sync_copy(ones_v, acc.at[dst_v.at[j]], add=True)
        return carry

    lax.fori_loop(0, nb, body, 0)
    plsc.subcore_barrier()
    pltpu.sync_copy(acc.at[pl.ds(s * rps, rps)],
                    degpart_hbm.at[pl.ds(c * n_pad + s * rps, rps)])


def _hop_body(nb, n_pad, g_hbm, src_hbm, dst_hbm, zeros_hbm, parts_hbm,
              src_v, dst_v, rows_a, acc, gsem):
    c = lax.axis_index("c")
    s = lax.axis_index("s")
    w = _wid(c, s)
    rps = n_pad // NS
    pltpu.sync_copy(zeros_hbm.at[pl.ds(s * rps, rps)],
                    acc.at[pl.ds(s * rps, rps)])
    pltpu.sync_copy(src_hbm.at[pl.ds(w * nb, nb)], src_v)
    pltpu.sync_copy(dst_hbm.at[pl.ds(w * nb, nb)], dst_v)
    plsc.subcore_barrier()

    def body(j, carry):
        pltpu.async_copy(g_hbm.at[src_v.at[j]], rows_a, gsem).wait()
        pltpu.sync_copy(rows_a, acc.at[dst_v.at[j]], add=True)
        return carry

    lax.fori_loop(0, nb, body, 0)
    plsc.subcore_barrier()
    pltpu.sync_copy(acc.at[pl.ds(s * rps, rps)],
                    parts_hbm.at[pl.ds(c * n_pad + s * rps, rps)])


def _scale0_tc_body(d0_ref, d1_ref, x_ref, g0_ref, nrm_ref, inv_ref):
    deg = jnp.maximum(d0_ref[...] + d1_ref[...], 1.0)
    nrm = lax.rsqrt(deg)
    nrm_ref[...] = nrm
    inv_ref[...] = 1.0 / deg
    g0_ref[...] = x_ref[...] * nrm


def _combine_tc_body(p0_ref, p1_ref, inv_ref, g_ref):
    g_ref[...] = (p0_ref[...] + p1_ref[...]) * inv_ref[...]


def _mm_body(p0_ref, p1_ref, n_ref, w_ref, o_ref):
    h = (p0_ref[...] + p1_ref[...]) * n_ref[...]
    o_ref[...] = lax.dot_general(h, w_ref[...], (((1,), (1,)), ((), ())),
                                 preferred_element_type=jnp.float32)


@jax.jit
def kernel(x, edge_index, W):
    n, d = x.shape
    e = edge_index.shape[1]
    n_pad = ((n + 511) // 512) * 512
    nb = (e + NW * EB - 1) // (NW * EB)
    nb = ((nb + 7) // 8) * 8
    e_pad = NW * nb * EB
    npad_extra = n_pad - n

    src = edge_index[0]
    dst = edge_index[1]
    pe = e_pad - e
    pad_src = jnp.arange(pe, dtype=jnp.int32) % jnp.int32(n)
    pad_dst = jnp.int32(n) + jnp.arange(pe, dtype=jnp.int32) % jnp.int32(npad_extra)
    srcp = jnp.concatenate([src, pad_src]).reshape(NW * nb, EB)
    dstp = jnp.concatenate([dst, pad_dst]).reshape(NW * nb, EB)
    xp = jnp.pad(x, ((0, npad_extra), (0, 0)))
    zeros2d = jnp.zeros((n_pad, d), jnp.float32)

    mesh = plsc.VectorSubcoreMesh(core_axis_name="c", subcore_axis_name="s")

    deg_call = pl.kernel(
        functools.partial(_deg_body, nb, n_pad),
        out_type=jax.ShapeDtypeStruct((NC * n_pad,), jnp.float32),
        mesh=mesh,
        compiler_params=pltpu.CompilerParams(needs_layout_passes=False),
        scratch_types=[
            pltpu.VMEM((nb, EB), jnp.int32),
            pltpu.VMEM((EB,), jnp.float32),
            pltpu.VMEM((128,), jnp.float32),
            pltpu.VMEM_SHARED((n_pad,), jnp.float32),
        ],
    )
    degpart = deg_call(dstp)

    blk = 2048
    scale0_call = pl.pallas_call(
        _scale0_tc_body,
        grid=(n_pad // blk,),
        in_specs=[
            pl.BlockSpec((blk, 1), lambda i: (i, 0)),
            pl.BlockSpec((blk, 1), lambda i: (i, 0)),
            pl.BlockSpec((blk, d), lambda i: (i, 0)),
        ],
        out_specs=(
            pl.BlockSpec((blk, d), lambda i: (i, 0)),
            pl.BlockSpec((blk, 1), lambda i: (i, 0)),
            pl.BlockSpec((blk, 1), lambda i: (i, 0)),
        ),
        out_shape=(
            jax.ShapeDtypeStruct((n_pad, d), jnp.float32),
            jax.ShapeDtypeStruct((n_pad, 1), jnp.float32),
            jax.ShapeDtypeStruct((n_pad, 1), jnp.float32),
        ),
    )
    g0, nrm2d, inv2d = scale0_call(degpart[:n_pad].reshape(n_pad, 1),
                                   degpart[n_pad:].reshape(n_pad, 1), xp)

    hop_call = pl.kernel(
        functools.partial(_hop_body, nb, n_pad),
        out_type=jax.ShapeDtypeStruct((NC * n_pad, d), jnp.float32),
        mesh=mesh,
        compiler_params=pltpu.CompilerParams(needs_layout_passes=False),
        scratch_types=[
            pltpu.VMEM((nb, EB), jnp.int32),
            pltpu.VMEM((nb, EB), jnp.int32),
            pltpu.VMEM((EB, d), jnp.float32),
            pltpu.VMEM_SHARED((n_pad, d), jnp.float32),
            pltpu.SemaphoreType.DMA,
        ],
    )
    parts1 = hop_call(g0, srcp, dstp, zeros2d)

    nblk = n_pad // blk
    combine_call = pl.pallas_call(
        _combine_tc_body,
        grid=(nblk,),
        in_specs=[
            pl.BlockSpec((blk, d), lambda i: (i, 0)),
            pl.BlockSpec((blk, d), lambda i: (i + nblk, 0)),
            pl.BlockSpec((blk, 1), lambda i: (i, 0)),
        ],
        out_specs=pl.BlockSpec((blk, d), lambda i: (i, 0)),
        out_shape=jax.ShapeDtypeStruct((n_pad, d), jnp.float32),
    )
    g1 = combine_call(parts1, parts1, inv2d)

    parts2 = hop_call(g1, srcp, dstp, zeros2d)

    mm_call = pl.pallas_call(
        _mm_body,
        grid=(nblk,),
        in_specs=[
            pl.BlockSpec((blk, d), lambda i: (i, 0)),
            pl.BlockSpec((blk, d), lambda i: (i + nblk, 0)),
            pl.BlockSpec((blk, 1), lambda i: (i, 0)),
            pl.BlockSpec((d, d), lambda i: (0, 0)),
        ],
        out_specs=pl.BlockSpec((blk, d), lambda i: (i, 0)),
        out_shape=jax.ShapeDtypeStruct((n_pad, d), jnp.float32),
    )
    out = mm_call(parts2, parts2, nrm2d, W)
    return out[:n]
